# Initial kernel scaffold; baseline (speedup 1.0000x reference)
import jax
import jax.numpy as jnp
from jax.experimental import pallas as pl


def kernel(x, Wq, bq, Wk, bk, Wv, bv, Wo, bo, Pq, bpq, Pk, bpk, Pv, bpv, Po, bpo, Wf, Wg, bg):
    raise NotImplementedError("write your pallas kernel here")


# speedup vs baseline: 5.9911x; 5.9911x over previous
import functools
import math

import jax
import jax.numpy as jnp
from jax import lax
from jax.experimental import pallas as pl

_HIGH = lax.Precision.DEFAULT


def _dot(a, b, dims, prec=_HIGH):
    return lax.dot_general(a, b, dimension_numbers=(dims, ((), ())),
                           precision=prec, preferred_element_type=jnp.float32)


def _proj6_body(x_ref, w_ref, b_ref, o_ref, *, nh, hd):
    full = _dot(x_ref[...], w_ref[0], (((1,), (0,)))) + b_ref[0]
    for h in range(nh):
        o_ref[0, 0, h] = full[:, h * hd:(h + 1) * hd]


def _proj6(x_flat, W6, b6, B, S, nh, hd, rb):
    n, d = x_flat.shape
    nblk = n // rb
    sb = S // rb
    return pl.pallas_call(
        functools.partial(_proj6_body, nh=nh, hd=hd),
        grid=(6, nblk),
        in_specs=[
            pl.BlockSpec((rb, d), lambda w, i: (i, 0)),
            pl.BlockSpec((1, d, d), lambda w, i: (w, 0, 0)),
            pl.BlockSpec((1, 1, d), lambda w, i: (w, 0, 0)),
        ],
        out_specs=pl.BlockSpec((1, 1, nh, rb, hd),
                               lambda w, i, sb=sb: (w, i // sb, 0, i % sb, 0)),
        out_shape=jax.ShapeDtypeStruct((6, B, nh, S, hd), jnp.float32),
    )(x_flat, W6, b6)


def _phi_body(a_ref, wf_ref, o_ref):
    a = _dot(a_ref[0, 0], wf_ref[...], (((1,), (0,))))
    o_ref[0, 0] = jnp.where(a > 0, a + 1.0, jnp.exp(a))


def _phi(qk2, Wf, rb):
    two, bh, S, hd = qk2.shape
    m = Wf.shape[1]
    return pl.pallas_call(
        _phi_body,
        grid=(two, bh, S // rb),
        in_specs=[
            pl.BlockSpec((1, 1, rb, hd), lambda j, g, i: (j, g, i, 0)),
            pl.BlockSpec((hd, m), lambda j, g, i: (0, 0)),
        ],
        out_specs=pl.BlockSpec((1, 1, rb, m), lambda j, g, i: (j, g, i, 0)),
        out_shape=jax.ShapeDtypeStruct((two, bh, S, m), jnp.float32),
    )(qk2, Wf)


def _attn_body(q_ref, k_ref, v_ref, o_ref, *, k_keep, scale):
    s = _dot(q_ref[0], k_ref[0], (((1,), (1,)))) * scale
    m = jnp.max(s, axis=1, keepdims=True)
    bits = lax.bitcast_convert_type(s, jnp.int32)
    keys = jnp.where(bits >= 0, bits, bits ^ jnp.int32(0x7FFFFFFF))
    lo = jnp.min(keys, axis=1, keepdims=True)
    hi = jnp.max(keys, axis=1, keepdims=True) + 1

    def body(_, carry):
        lo, hi = carry
        mid = (lo >> 1) + (hi >> 1) + (lo & hi & 1)
        cnt = jnp.sum((keys >= mid).astype(jnp.int32), axis=1, keepdims=True)
        ge = cnt >= k_keep
        return jnp.where(ge, mid, lo), jnp.where(ge, hi, mid)

    lo, hi = lax.fori_loop(0, 32, body, (lo, hi))
    p = jnp.where(keys >= lo, jnp.exp(s - m), 0.0)
    denom = jnp.sum(p, axis=1, keepdims=True)
    o_ref[0] = _dot(p, v_ref[0], (((1,), (0,)))) / denom


def _attn(q, k, v, k_keep, hd, bq):
    bh, S, _ = q.shape
    return pl.pallas_call(
        functools.partial(_attn_body, k_keep=k_keep, scale=1.0 / math.sqrt(hd)),
        grid=(bh, S // bq),
        in_specs=[
            pl.BlockSpec((1, bq, hd), lambda g, i: (g, i, 0)),
            pl.BlockSpec((1, S, hd), lambda g, i: (g, 0, 0)),
            pl.BlockSpec((1, S, hd), lambda g, i: (g, 0, 0)),
        ],
        out_specs=pl.BlockSpec((1, bq, hd), lambda g, i: (g, i, 0)),
        out_shape=jax.ShapeDtypeStruct((bh, S, hd), jnp.float32),
    )(q, k, v)


def _perf_body(pq_ref, pk_ref, v_ref, o_ref):
    pk = pk_ref[0]
    pq = pq_ref[0]
    kv = _dot(pk, v_ref[0], (((0,), (0,))))
    ksum = _dot(jnp.ones((1, pk.shape[0]), jnp.float32), pk, (((1,), (0,))))
    z = jnp.sum(pq * ksum, axis=1, keepdims=True)
    num = _dot(pq, kv, (((1,), (0,))))
    o_ref[0] = num / (z + 1e-6)


def _perf(phi_q, phi_k, v2):
    bh, S, m = phi_q.shape
    hd = v2.shape[2]
    return pl.pallas_call(
        _perf_body,
        grid=(bh,),
        in_specs=[
            pl.BlockSpec((1, S, m), lambda g: (g, 0, 0)),
            pl.BlockSpec((1, S, m), lambda g: (g, 0, 0)),
            pl.BlockSpec((1, S, hd), lambda g: (g, 0, 0)),
        ],
        out_specs=pl.BlockSpec((1, S, hd), lambda g: (g, 0, 0)),
        out_shape=jax.ShapeDtypeStruct((bh, S, hd), jnp.float32),
    )(phi_q, phi_k, v2)


def _outp_body(sp_ref, pf_ref, wo_ref, bo_ref, po_ref, bpo_ref, g_ref, o_ref,
               *, nh):
    sp = jnp.concatenate([sp_ref[0, h] for h in range(nh)], axis=1)
    pf = jnp.concatenate([pf_ref[0, h] for h in range(nh)], axis=1)
    so = _dot(sp, wo_ref[...], (((1,), (0,)))) + bo_ref[0]
    po = _dot(pf, po_ref[...], (((1,), (0,)))) + bpo_ref[0]
    g = g_ref[0]
    o_ref[0] = so * g[:, 0:1] + po * g[:, 1:2]


def _outp(spc, pfc, Wo, bo, Po, bpo, gate, rb):
    B, nh, S, hd = spc.shape
    d = Wo.shape[0]
    sb = S // rb
    return pl.pallas_call(
        functools.partial(_outp_body, nh=nh),
        grid=(B * sb,),
        in_specs=[
            pl.BlockSpec((1, nh, rb, hd), lambda i, sb=sb: (i // sb, 0, i % sb, 0)),
            pl.BlockSpec((1, nh, rb, hd), lambda i, sb=sb: (i // sb, 0, i % sb, 0)),
            pl.BlockSpec((d, d), lambda i: (0, 0)),
            pl.BlockSpec((1, d), lambda i: (0, 0)),
            pl.BlockSpec((d, d), lambda i: (0, 0)),
            pl.BlockSpec((1, d), lambda i: (0, 0)),
            pl.BlockSpec((1, 1, 2), lambda i, sb=sb: (i // sb, 0, 0)),
        ],
        out_specs=pl.BlockSpec((1, rb, d), lambda i, sb=sb: (i // sb, i % sb, 0)),
        out_shape=jax.ShapeDtypeStruct((B, S, d), jnp.float32),
    )(spc, pfc, Wo, bo.reshape(1, d), Po, bpo.reshape(1, d), gate)


def kernel(x, Wq, bq, Wk, bk, Wv, bv, Wo, bo, Pq, bpq, Pk, bpk, Pv, bpv,
           Po, bpo, Wf, Wg, bg):
    B, S, D = x.shape
    hd, m = Wf.shape
    nh = D // hd
    k_keep = 64

    W6 = jnp.stack([Wq, Wk, Wv, Pq, Pk, Pv])
    b6 = jnp.stack([bq, bk, bv, bpq, bpk, bpv]).reshape(6, 1, D)

    out6 = _proj6(x.reshape(B * S, D), W6, b6, B, S, nh, hd, rb=256)
    qkv = out6.reshape(6, B * nh, S, hd)
    q, k, v = qkv[0], qkv[1], qkv[2]
    v2 = qkv[5]

    phis = _phi(out6[3:5].reshape(2, B * nh, S, hd), Wf, rb=512)
    phi_q, phi_k = phis[0], phis[1]

    spc = _attn(q, k, v, k_keep, hd, bq=256)
    pfc = _perf(phi_q, phi_k, v2)

    gate = jax.nn.softmax(jnp.mean(x, axis=1) @ Wg + bg, axis=-1)

    return _outp(spc.reshape(B, nh, S, hd), pfc.reshape(B, nh, S, hd),
                 Wo, bo, Po, bpo, gate.reshape(B, 1, 2), rb=256)


# speedup vs baseline: 6.8079x; 1.1363x over previous
import functools
import math

import jax
import jax.numpy as jnp
from jax import lax
from jax.experimental import pallas as pl


def _dot(a, b, dims):
    return lax.dot_general(a, b, dimension_numbers=(dims, ((), ())),
                           preferred_element_type=jnp.float32)


def _proj_body(x_ref, w_ref, b_ref, o_ref, *, nh, hd):
    full = _dot(x_ref[...], w_ref[0], (((1,), (0,)))) + b_ref[0]
    for h in range(nh):
        o_ref[0, 0, h] = full[:, h * hd:(h + 1) * hd].astype(o_ref.dtype)


def _proj(x_flat, Ws, bs, B, S, nh, hd, rb, out_dtype):
    n, d = x_flat.shape
    nw = Ws.shape[0]
    nblk = n // rb
    sb = S // rb
    return pl.pallas_call(
        functools.partial(_proj_body, nh=nh, hd=hd),
        grid=(nw, nblk),
        in_specs=[
            pl.BlockSpec((rb, d), lambda w, i: (i, 0)),
            pl.BlockSpec((1, d, d), lambda w, i: (w, 0, 0)),
            pl.BlockSpec((1, 1, d), lambda w, i: (w, 0, 0)),
        ],
        out_specs=pl.BlockSpec((1, 1, nh, rb, hd),
                               lambda w, i, sb=sb: (w, i // sb, 0, i % sb, 0)),
        out_shape=jax.ShapeDtypeStruct((nw, B, nh, S, hd), out_dtype),
    )(x_flat, Ws, bs)


def _phi_body(a_ref, wf_ref, o_ref):
    a = _dot(a_ref[0, 0], wf_ref[...], (((1,), (0,))))
    o_ref[0, 0] = jnp.where(a > 0, a + 1.0, jnp.exp(a)).astype(o_ref.dtype)


def _phi(qk2, Wf, rb):
    two, bh, S, hd = qk2.shape
    m = Wf.shape[1]
    return pl.pallas_call(
        _phi_body,
        grid=(two, bh, S // rb),
        in_specs=[
            pl.BlockSpec((1, 1, rb, hd), lambda j, g, i: (j, g, i, 0)),
            pl.BlockSpec((hd, m), lambda j, g, i: (0, 0)),
        ],
        out_specs=pl.BlockSpec((1, 1, rb, m), lambda j, g, i: (j, g, i, 0)),
        out_shape=jax.ShapeDtypeStruct((two, bh, S, m), jnp.bfloat16),
    )(qk2, Wf)


def _attn_body(q_ref, k_ref, v_ref, o_ref, *, k_keep, scale):
    s = _dot(q_ref[0], k_ref[0], (((1,), (1,)))) * scale
    m = jnp.max(s, axis=1, keepdims=True)
    bits = lax.bitcast_convert_type(s, jnp.int32)
    keys = jnp.where(bits >= 0, bits, bits ^ jnp.int32(0x7FFFFFFF))
    lo = jnp.min(keys, axis=1, keepdims=True)
    hi = jnp.max(keys, axis=1, keepdims=True) + 1

    def cond(carry):
        lo, hi = carry
        return jnp.any((hi - lo) > 1)

    def body(carry):
        lo, hi = carry
        mid = (lo >> 1) + (hi >> 1) + (lo & hi & 1)
        cnt = jnp.sum((keys >= mid).astype(jnp.int32), axis=1, keepdims=True)
        ge = cnt >= k_keep
        eq = cnt == k_keep
        return (jnp.where(ge, mid, lo),
                jnp.where(eq, mid, jnp.where(ge, hi, mid)))

    lo, hi = lax.while_loop(cond, body, (lo, hi))
    p = jnp.where(keys >= lo, jnp.exp(s - m), 0.0)
    denom = jnp.sum(p, axis=1, keepdims=True)
    o_ref[0] = _dot(p.astype(jnp.bfloat16), v_ref[0], (((1,), (0,)))) / denom


def _attn(q, k, v, k_keep, hd, bq):
    bh, S, _ = q.shape
    return pl.pallas_call(
        functools.partial(_attn_body, k_keep=k_keep, scale=1.0 / math.sqrt(hd)),
        grid=(bh, S // bq),
        in_specs=[
            pl.BlockSpec((1, bq, hd), lambda g, i: (g, i, 0)),
            pl.BlockSpec((1, S, hd), lambda g, i: (g, 0, 0)),
            pl.BlockSpec((1, S, hd), lambda g, i: (g, 0, 0)),
        ],
        out_specs=pl.BlockSpec((1, bq, hd), lambda g, i: (g, i, 0)),
        out_shape=jax.ShapeDtypeStruct((bh, S, hd), jnp.float32),
    )(q, k, v)


def _perf_body(pq_ref, pk_ref, v_ref, o_ref):
    pk = pk_ref[0]
    pq = pq_ref[0]
    kv = _dot(pk, v_ref[0], (((0,), (0,))))
    ksum = _dot(jnp.ones((1, pk.shape[0]), pk.dtype), pk, (((1,), (0,))))
    z = jnp.sum(pq.astype(jnp.float32) * ksum, axis=1, keepdims=True)
    num = _dot(pq, kv.astype(jnp.bfloat16), (((1,), (0,))))
    o_ref[0] = num / (z + 1e-6)


def _perf(phi_q, phi_k, v2):
    bh, S, m = phi_q.shape
    hd = v2.shape[2]
    return pl.pallas_call(
        _perf_body,
        grid=(bh,),
        in_specs=[
            pl.BlockSpec((1, S, m), lambda g: (g, 0, 0)),
            pl.BlockSpec((1, S, m), lambda g: (g, 0, 0)),
            pl.BlockSpec((1, S, hd), lambda g: (g, 0, 0)),
        ],
        out_specs=pl.BlockSpec((1, S, hd), lambda g: (g, 0, 0)),
        out_shape=jax.ShapeDtypeStruct((bh, S, hd), jnp.float32),
    )(phi_q, phi_k, v2)


def _outp_body(sp_ref, pf_ref, wo_ref, bo_ref, po_ref, bpo_ref, g_ref, o_ref,
               *, nh):
    sp = jnp.concatenate([sp_ref[0, h] for h in range(nh)], axis=1)
    pf = jnp.concatenate([pf_ref[0, h] for h in range(nh)], axis=1)
    so = _dot(sp.astype(jnp.bfloat16), wo_ref[...], (((1,), (0,)))) + bo_ref[0]
    po = _dot(pf.astype(jnp.bfloat16), po_ref[...], (((1,), (0,)))) + bpo_ref[0]
    g = g_ref[0]
    o_ref[0] = so * g[:, 0:1] + po * g[:, 1:2]


def _outp(spc, pfc, Wo, bo, Po, bpo, gate, rb):
    B, nh, S, hd = spc.shape
    d = Wo.shape[0]
    sb = S // rb
    return pl.pallas_call(
        functools.partial(_outp_body, nh=nh),
        grid=(B * sb,),
        in_specs=[
            pl.BlockSpec((1, nh, rb, hd), lambda i, sb=sb: (i // sb, 0, i % sb, 0)),
            pl.BlockSpec((1, nh, rb, hd), lambda i, sb=sb: (i // sb, 0, i % sb, 0)),
            pl.BlockSpec((d, d), lambda i: (0, 0)),
            pl.BlockSpec((1, d), lambda i: (0, 0)),
            pl.BlockSpec((d, d), lambda i: (0, 0)),
            pl.BlockSpec((1, d), lambda i: (0, 0)),
            pl.BlockSpec((1, 1, 2), lambda i, sb=sb: (i // sb, 0, 0)),
        ],
        out_specs=pl.BlockSpec((1, rb, d), lambda i, sb=sb: (i // sb, i % sb, 0)),
        out_shape=jax.ShapeDtypeStruct((B, S, d), jnp.float32),
    )(spc, pfc, Wo.astype(jnp.bfloat16), bo.reshape(1, d),
      Po.astype(jnp.bfloat16), bpo.reshape(1, d), gate)


def kernel(x, Wq, bq, Wk, bk, Wv, bv, Wo, bo, Pq, bpq, Pk, bpk, Pv, bpv,
           Po, bpo, Wf, Wg, bg):
    B, S, D = x.shape
    hd, m = Wf.shape
    nh = D // hd
    k_keep = 64

    x_flat = x.reshape(B * S, D)
    Wqk = jnp.stack([Wq, Wk])
    bqk = jnp.stack([bq, bk]).reshape(2, 1, D)
    qk = _proj(x_flat, Wqk, bqk, B, S, nh, hd, 256, jnp.float32)
    qk = qk.reshape(2, B * nh, S, hd)

    W4 = jnp.stack([Wv, Pq, Pk, Pv]).astype(jnp.bfloat16)
    b4 = jnp.stack([bv, bpq, bpk, bpv]).reshape(4, 1, D)
    p4 = _proj(x_flat.astype(jnp.bfloat16), W4, b4, B, S, nh, hd, 256,
               jnp.bfloat16)
    p4 = p4.reshape(4, B * nh, S, hd)

    phis = _phi(p4[1:3], Wf.astype(jnp.bfloat16), rb=512)

    spc = _attn(qk[0], qk[1], p4[0], k_keep, hd, bq=256)
    pfc = _perf(phis[0], phis[1], p4[3])

    gate = jax.nn.softmax(jnp.mean(x, axis=1) @ Wg + bg, axis=-1)

    return _outp(spc.reshape(B, nh, S, hd), pfc.reshape(B, nh, S, hd),
                 Wo, bo, Po, bpo, gate.reshape(B, 1, 2), rb=256)


# speedup vs baseline: 7.4033x; 1.0874x over previous
import functools
import math

import jax
import jax.numpy as jnp
from jax import lax
from jax.experimental import pallas as pl


def _dot(a, b, dims):
    return lax.dot_general(a, b, dimension_numbers=(dims, ((), ())),
                           preferred_element_type=jnp.float32)


def _proj_body(x_ref, w_ref, b_ref, o_ref, *, nh, hd):
    full = _dot(x_ref[...], w_ref[0], (((1,), (0,)))) + b_ref[0]
    for h in range(nh):
        o_ref[0, 0, h] = full[:, h * hd:(h + 1) * hd].astype(o_ref.dtype)


def _proj(x_flat, Ws, bs, B, S, nh, hd, rb, out_dtype):
    n, d = x_flat.shape
    nw = Ws.shape[0]
    nblk = n // rb
    sb = S // rb
    return pl.pallas_call(
        functools.partial(_proj_body, nh=nh, hd=hd),
        grid=(nw, nblk),
        in_specs=[
            pl.BlockSpec((rb, d), lambda w, i: (i, 0)),
            pl.BlockSpec((1, d, d), lambda w, i: (w, 0, 0)),
            pl.BlockSpec((1, 1, d), lambda w, i: (w, 0, 0)),
        ],
        out_specs=pl.BlockSpec((1, 1, nh, rb, hd),
                               lambda w, i, sb=sb: (w, i // sb, 0, i % sb, 0)),
        out_shape=jax.ShapeDtypeStruct((nw, B, nh, S, hd), out_dtype),
    )(x_flat, Ws, bs)


def _phi_body(a_ref, wf_ref, o_ref):
    a = _dot(a_ref[0, 0], wf_ref[...], (((1,), (0,))))
    o_ref[0, 0] = jnp.where(a > 0, a + 1.0, jnp.exp(a)).astype(o_ref.dtype)


def _phi(qk2, Wf, rb):
    two, bh, S, hd = qk2.shape
    m = Wf.shape[1]
    return pl.pallas_call(
        _phi_body,
        grid=(two, bh, S // rb),
        in_specs=[
            pl.BlockSpec((1, 1, rb, hd), lambda j, g, i: (j, g, i, 0)),
            pl.BlockSpec((hd, m), lambda j, g, i: (0, 0)),
        ],
        out_specs=pl.BlockSpec((1, 1, rb, m), lambda j, g, i: (j, g, i, 0)),
        out_shape=jax.ShapeDtypeStruct((two, bh, S, m), jnp.bfloat16),
    )(qk2, Wf)


def _attn_body(q_ref, k_ref, v_ref, o_ref, *, k_keep, scale):
    s = _dot(q_ref[0], k_ref[0], (((1,), (1,)))) * scale
    m = jnp.max(s, axis=1, keepdims=True)
    bits = lax.bitcast_convert_type(s, jnp.int32)
    keys = jnp.where(bits >= 0, bits, bits ^ jnp.int32(0x7FFFFFFF))
    lo = jnp.min(keys, axis=1, keepdims=True)
    hi = jnp.max(keys, axis=1, keepdims=True) + 1
    def cond(carry):
        lo, hi = carry
        return jnp.any((hi - lo) > 1)

    def body(carry):
        lo, hi = carry
        mid = (lo >> 1) + (hi >> 1) + (lo & hi & 1)
        cnt = jnp.sum((keys >= mid).astype(jnp.int32), axis=1, keepdims=True)
        ge = cnt >= k_keep
        eq = cnt == k_keep
        return (jnp.where(ge, mid, lo),
                jnp.where(eq, mid, jnp.where(ge, hi, mid)))

    lo, hi = lax.while_loop(cond, body, (lo, hi))
    p = jnp.where(keys >= lo, jnp.exp(s - m), 0.0)
    denom = jnp.sum(p, axis=1, keepdims=True)
    o_ref[0] = _dot(p.astype(jnp.bfloat16), v_ref[0], (((1,), (0,)))) / denom


def _attn(q, k, v, k_keep, hd, bq):
    bh, S, _ = q.shape
    return pl.pallas_call(
        functools.partial(_attn_body, k_keep=k_keep, scale=1.0 / math.sqrt(hd)),
        grid=(bh, S // bq),
        in_specs=[
            pl.BlockSpec((1, bq, hd), lambda g, i: (g, i, 0)),
            pl.BlockSpec((1, S, hd), lambda g, i: (g, 0, 0)),
            pl.BlockSpec((1, S, hd), lambda g, i: (g, 0, 0)),
        ],
        out_specs=pl.BlockSpec((1, bq, hd), lambda g, i: (g, i, 0)),
        out_shape=jax.ShapeDtypeStruct((bh, S, hd), jnp.float32),
    )(q, k, v)


def _perf_body(pq_ref, pk_ref, v_ref, o_ref):
    pk = pk_ref[0]
    pq = pq_ref[0]
    kv = _dot(pk, v_ref[0], (((0,), (0,))))
    ksum = _dot(jnp.ones((1, pk.shape[0]), pk.dtype), pk, (((1,), (0,))))
    z = jnp.sum(pq.astype(jnp.float32) * ksum, axis=1, keepdims=True)
    num = _dot(pq, kv.astype(jnp.bfloat16), (((1,), (0,))))
    o_ref[0] = num / (z + 1e-6)


def _perf(phi_q, phi_k, v2):
    bh, S, m = phi_q.shape
    hd = v2.shape[2]
    return pl.pallas_call(
        _perf_body,
        grid=(bh,),
        in_specs=[
            pl.BlockSpec((1, S, m), lambda g: (g, 0, 0)),
            pl.BlockSpec((1, S, m), lambda g: (g, 0, 0)),
            pl.BlockSpec((1, S, hd), lambda g: (g, 0, 0)),
        ],
        out_specs=pl.BlockSpec((1, S, hd), lambda g: (g, 0, 0)),
        out_shape=jax.ShapeDtypeStruct((bh, S, hd), jnp.float32),
    )(phi_q, phi_k, v2)


def _outp_body(sp_ref, pf_ref, wo_ref, bo_ref, po_ref, bpo_ref, g_ref, o_ref,
               *, nh):
    sp = jnp.concatenate([sp_ref[0, h] for h in range(nh)], axis=1)
    pf = jnp.concatenate([pf_ref[0, h] for h in range(nh)], axis=1)
    so = _dot(sp.astype(jnp.bfloat16), wo_ref[...], (((1,), (0,)))) + bo_ref[0]
    po = _dot(pf.astype(jnp.bfloat16), po_ref[...], (((1,), (0,)))) + bpo_ref[0]
    g = g_ref[0]
    o_ref[0] = so * g[:, 0:1] + po * g[:, 1:2]


def _outp(spc, pfc, Wo, bo, Po, bpo, gate, rb):
    B, nh, S, hd = spc.shape
    d = Wo.shape[0]
    sb = S // rb
    return pl.pallas_call(
        functools.partial(_outp_body, nh=nh),
        grid=(B * sb,),
        in_specs=[
            pl.BlockSpec((1, nh, rb, hd), lambda i, sb=sb: (i // sb, 0, i % sb, 0)),
            pl.BlockSpec((1, nh, rb, hd), lambda i, sb=sb: (i // sb, 0, i % sb, 0)),
            pl.BlockSpec((d, d), lambda i: (0, 0)),
            pl.BlockSpec((1, d), lambda i: (0, 0)),
            pl.BlockSpec((d, d), lambda i: (0, 0)),
            pl.BlockSpec((1, d), lambda i: (0, 0)),
            pl.BlockSpec((1, 1, 2), lambda i, sb=sb: (i // sb, 0, 0)),
        ],
        out_specs=pl.BlockSpec((1, rb, d), lambda i, sb=sb: (i // sb, i % sb, 0)),
        out_shape=jax.ShapeDtypeStruct((B, S, d), jnp.float32),
    )(spc, pfc, Wo.astype(jnp.bfloat16), bo.reshape(1, d),
      Po.astype(jnp.bfloat16), bpo.reshape(1, d), gate)


def kernel(x, Wq, bq, Wk, bk, Wv, bv, Wo, bo, Pq, bpq, Pk, bpk, Pv, bpv,
           Po, bpo, Wf, Wg, bg):
    B, S, D = x.shape
    hd, m = Wf.shape
    nh = D // hd
    k_keep = 64

    x_flat = x.reshape(B * S, D)
    Wqk = jnp.stack([Wq, Wk])
    bqk = jnp.stack([bq, bk]).reshape(2, 1, D)
    qk = _proj(x_flat, Wqk, bqk, B, S, nh, hd, 512, jnp.float32)
    qk = qk.reshape(2, B * nh, S, hd)

    W4 = jnp.stack([Wv, Pq, Pk, Pv]).astype(jnp.bfloat16)
    b4 = jnp.stack([bv, bpq, bpk, bpv]).reshape(4, 1, D)
    p4 = _proj(x_flat.astype(jnp.bfloat16), W4, b4, B, S, nh, hd, 512,
               jnp.bfloat16)
    p4 = p4.reshape(4, B * nh, S, hd)

    phis = _phi(p4[1:3], Wf.astype(jnp.bfloat16), rb=512)

    spc = _attn(qk[0], qk[1], p4[0], k_keep, hd, bq=512)
    pfc = _perf(phis[0], phis[1], p4[3])

    gate = jax.nn.softmax(jnp.mean(x, axis=1) @ Wg + bg, axis=-1)

    return _outp(spc.reshape(B, nh, S, hd), pfc.reshape(B, nh, S, hd),
                 Wo, bo, Po, bpo, gate.reshape(B, 1, 2), rb=512)


# speedup vs baseline: 7.5474x; 1.0195x over previous
import functools
import math

import jax
import jax.numpy as jnp
from jax import lax
from jax.experimental import pallas as pl


def _dot(a, b, dims):
    return lax.dot_general(a, b, dimension_numbers=(dims, ((), ())),
                           preferred_element_type=jnp.float32)


def _proj_body(x_ref, w_ref, b_ref, o_ref, *, nh, hd):
    full = _dot(x_ref[...], w_ref[0], (((1,), (0,)))) + b_ref[0]
    for h in range(nh):
        o_ref[0, 0, h] = full[:, h * hd:(h + 1) * hd].astype(o_ref.dtype)


def _proj(x_flat, Ws, bs, B, S, nh, hd, rb, out_dtype):
    n, d = x_flat.shape
    nw = Ws.shape[0]
    nblk = n // rb
    sb = S // rb
    return pl.pallas_call(
        functools.partial(_proj_body, nh=nh, hd=hd),
        grid=(nw, nblk),
        in_specs=[
            pl.BlockSpec((rb, d), lambda w, i: (i, 0)),
            pl.BlockSpec((1, d, d), lambda w, i: (w, 0, 0)),
            pl.BlockSpec((1, 1, d), lambda w, i: (w, 0, 0)),
        ],
        out_specs=pl.BlockSpec((1, 1, nh, rb, hd),
                               lambda w, i, sb=sb: (w, i // sb, 0, i % sb, 0)),
        out_shape=jax.ShapeDtypeStruct((nw, B, nh, S, hd), out_dtype),
    )(x_flat, Ws, bs)


def _phi_body(a_ref, wf_ref, o_ref):
    a = _dot(a_ref[0, 0], wf_ref[...], (((1,), (0,))))
    o_ref[0, 0] = jnp.where(a > 0, a + 1.0, jnp.exp(a)).astype(o_ref.dtype)


def _phi(qk2, Wf, rb):
    two, bh, S, hd = qk2.shape
    m = Wf.shape[1]
    return pl.pallas_call(
        _phi_body,
        grid=(two, bh, S // rb),
        in_specs=[
            pl.BlockSpec((1, 1, rb, hd), lambda j, g, i: (j, g, i, 0)),
            pl.BlockSpec((hd, m), lambda j, g, i: (0, 0)),
        ],
        out_specs=pl.BlockSpec((1, 1, rb, m), lambda j, g, i: (j, g, i, 0)),
        out_shape=jax.ShapeDtypeStruct((two, bh, S, m), jnp.bfloat16),
    )(qk2, Wf)


def _attn_body(q_ref, k_ref, v_ref, o_ref, *, k_keep, scale):
    s = _dot(q_ref[0], k_ref[0], (((1,), (1,)))) * scale
    m = jnp.max(s, axis=1, keepdims=True)
    bits = lax.bitcast_convert_type(s, jnp.int32)
    keys = jnp.where(bits >= 0, bits, bits ^ jnp.int32(0x7FFFFFFF))
    lo = jnp.min(keys, axis=1, keepdims=True)
    hi = jnp.max(keys, axis=1, keepdims=True) + 1
    def cond(carry):
        lo, hi = carry
        return jnp.any((hi - lo) > 1)

    def body(carry):
        lo, hi = carry
        mid = (lo >> 1) + (hi >> 1) + (lo & hi & 1)
        cnt = jnp.sum((keys >= mid).astype(jnp.int32), axis=1, keepdims=True)
        ge = cnt >= k_keep
        eq = cnt == k_keep
        return (jnp.where(ge, mid, lo),
                jnp.where(eq, mid, jnp.where(ge, hi, mid)))

    lo, hi = lax.while_loop(cond, body, (lo, hi))
    p = jnp.where(keys >= lo, jnp.exp(s - m), 0.0)
    denom = jnp.sum(p, axis=1, keepdims=True)
    o_ref[0] = _dot(p.astype(jnp.bfloat16), v_ref[0], (((1,), (0,)))) / denom


def _attn(q, k, v, k_keep, hd, bq):
    bh, S, _ = q.shape
    return pl.pallas_call(
        functools.partial(_attn_body, k_keep=k_keep, scale=1.0 / math.sqrt(hd)),
        grid=(bh, S // bq),
        in_specs=[
            pl.BlockSpec((1, bq, hd), lambda g, i: (g, i, 0)),
            pl.BlockSpec((1, S, hd), lambda g, i: (g, 0, 0)),
            pl.BlockSpec((1, S, hd), lambda g, i: (g, 0, 0)),
        ],
        out_specs=pl.BlockSpec((1, bq, hd), lambda g, i: (g, i, 0)),
        out_shape=jax.ShapeDtypeStruct((bh, S, hd), jnp.float32),
    )(q, k, v)


def _perf_body(pq_ref, pk_ref, v_ref, o_ref):
    pk = pk_ref[0]
    pq = pq_ref[0]
    kv = _dot(pk, v_ref[0], (((0,), (0,))))
    ksum = _dot(jnp.ones((1, pk.shape[0]), pk.dtype), pk, (((1,), (0,))))
    z = jnp.sum(pq.astype(jnp.float32) * ksum, axis=1, keepdims=True)
    num = _dot(pq, kv.astype(jnp.bfloat16), (((1,), (0,))))
    o_ref[0] = num / (z + 1e-6)


def _perf(phi_q, phi_k, v2):
    bh, S, m = phi_q.shape
    hd = v2.shape[2]
    return pl.pallas_call(
        _perf_body,
        grid=(bh,),
        in_specs=[
            pl.BlockSpec((1, S, m), lambda g: (g, 0, 0)),
            pl.BlockSpec((1, S, m), lambda g: (g, 0, 0)),
            pl.BlockSpec((1, S, hd), lambda g: (g, 0, 0)),
        ],
        out_specs=pl.BlockSpec((1, S, hd), lambda g: (g, 0, 0)),
        out_shape=jax.ShapeDtypeStruct((bh, S, hd), jnp.float32),
    )(phi_q, phi_k, v2)


def _outp_body(sp_ref, pf_ref, wo_ref, bo_ref, po_ref, bpo_ref, g_ref, o_ref,
               *, nh):
    sp = jnp.concatenate([sp_ref[0, h] for h in range(nh)], axis=1)
    pf = jnp.concatenate([pf_ref[0, h] for h in range(nh)], axis=1)
    so = _dot(sp.astype(jnp.bfloat16), wo_ref[...], (((1,), (0,)))) + bo_ref[0]
    po = _dot(pf.astype(jnp.bfloat16), po_ref[...], (((1,), (0,)))) + bpo_ref[0]
    g = g_ref[0]
    o_ref[0] = so * g[:, 0:1] + po * g[:, 1:2]


def _outp(spc, pfc, Wo, bo, Po, bpo, gate, rb):
    B, nh, S, hd = spc.shape
    d = Wo.shape[0]
    sb = S // rb
    return pl.pallas_call(
        functools.partial(_outp_body, nh=nh),
        grid=(B * sb,),
        in_specs=[
            pl.BlockSpec((1, nh, rb, hd), lambda i, sb=sb: (i // sb, 0, i % sb, 0)),
            pl.BlockSpec((1, nh, rb, hd), lambda i, sb=sb: (i // sb, 0, i % sb, 0)),
            pl.BlockSpec((d, d), lambda i: (0, 0)),
            pl.BlockSpec((1, d), lambda i: (0, 0)),
            pl.BlockSpec((d, d), lambda i: (0, 0)),
            pl.BlockSpec((1, d), lambda i: (0, 0)),
            pl.BlockSpec((1, 1, 2), lambda i, sb=sb: (i // sb, 0, 0)),
        ],
        out_specs=pl.BlockSpec((1, rb, d), lambda i, sb=sb: (i // sb, i % sb, 0)),
        out_shape=jax.ShapeDtypeStruct((B, S, d), jnp.float32),
    )(spc, pfc, Wo.astype(jnp.bfloat16), bo.reshape(1, d),
      Po.astype(jnp.bfloat16), bpo.reshape(1, d), gate)


def kernel(x, Wq, bq, Wk, bk, Wv, bv, Wo, bo, Pq, bpq, Pk, bpk, Pv, bpv,
           Po, bpo, Wf, Wg, bg):
    B, S, D = x.shape
    hd, m = Wf.shape
    nh = D // hd
    k_keep = 64

    x_flat = x.reshape(B * S, D)
    Wqk = jnp.stack([Wq, Wk])
    bqk = jnp.stack([bq, bk]).reshape(2, 1, D)
    qk = _proj(x_flat, Wqk, bqk, B, S, nh, hd, 512, jnp.float32)
    qk = qk.reshape(2, B * nh, S, hd)

    W4 = jnp.stack([Wv, Pq, Pk, Pv]).astype(jnp.bfloat16)
    b4 = jnp.stack([bv, bpq, bpk, bpv]).reshape(4, 1, D)
    p4 = _proj(x_flat.astype(jnp.bfloat16), W4, b4, B, S, nh, hd, 512,
               jnp.bfloat16)
    p4 = p4.reshape(4, B * nh, S, hd)

    phis = _phi(p4[1:3], Wf.astype(jnp.bfloat16), rb=512)

    spc = _attn(qk[0], qk[1], p4[0], k_keep, hd, bq=1024)
    pfc = _perf(phis[0], phis[1], p4[3])

    gate = jax.nn.softmax(jnp.mean(x, axis=1) @ Wg + bg, axis=-1)

    return _outp(spc.reshape(B, nh, S, hd), pfc.reshape(B, nh, S, hd),
                 Wo, bo, Po, bpo, gate.reshape(B, 1, 2), rb=512)
